# Initial kernel scaffold; baseline (speedup 1.0000x reference)
import jax
import jax.numpy as jnp
from jax.experimental import pallas as pl


def kernel(token_ids, embed_weight):
    raise NotImplementedError("write your pallas kernel here")


# speedup vs baseline: 11.9278x; 11.9278x over previous
import functools

import jax
import jax.numpy as jnp
from jax import lax
from jax.experimental import pallas as pl
from jax.experimental.pallas import tpu as pltpu
from jax.experimental.pallas import tpu_sc as plsc

D = 64
B = 16384
L = 200
LANES = 16

NC = 2
NS = 16
NW = NC * NS
ROWS_PER_W = B // NW

G0, G1 = 96, 104


def _tec_kernel(tok_hbm, tab_hbm, out_hbm, ids_v, rows_v, e0_v, out_v, sem):
    wid = lax.axis_index("s") * NC + lax.axis_index("c")
    base = wid * ROWS_PER_W

    pltpu.sync_copy(tab_hbm.at[pl.ds(0, 1)], e0_v)
    lanes = lax.iota(jnp.int32, LANES)

    def row_body(i, carry):
        b = base + i
        pltpu.sync_copy(tok_hbm.at[pl.ds(b * L, L)], ids_v.at[pl.ds(0, L)])
        g0 = pltpu.async_copy(
            tab_hbm.at[ids_v.at[pl.ds(0, G0)]], rows_v.at[pl.ds(0, G0)], sem)
        g1 = pltpu.async_copy(
            tab_hbm.at[ids_v.at[pl.ds(G0, G1)]], rows_v.at[pl.ds(G0, G1)], sem)

        zacc = jnp.zeros((LANES,), jnp.int32)
        for k in range(12):
            v = ids_v[pl.ds(k * LANES, LANES)]
            zacc = zacc + plsc.all_reduce_population_count(v == 0)
        v = ids_v[pl.ds(192, LANES)]
        zacc = zacc + plsc.all_reduce_population_count((v == 0) & (lanes < 8))
        n0v = zacc.astype(jnp.float32)
        invv = 1.0 / jnp.maximum(float(L) - n0v, 1.0)

        g0.wait()
        g1.wait()

        def acc_body(j, accs):
            a0, a1, a2, a3 = accs
            r0 = j * 8
            for u in range(8):
                r = r0 + u
                a0 = a0 + rows_v[r, pl.ds(0, LANES)]
                a1 = a1 + rows_v[r, pl.ds(16, LANES)]
                a2 = a2 + rows_v[r, pl.ds(32, LANES)]
                a3 = a3 + rows_v[r, pl.ds(48, LANES)]
            return (a0, a1, a2, a3)

        z = jnp.zeros((LANES,), jnp.float32)
        accs = lax.fori_loop(0, L // 8, acc_body, (z, z, z, z))
        for c in range(4):
            e0c = e0_v[0, pl.ds(c * LANES, LANES)]
            out_v[pl.ds(c * LANES, LANES)] = (accs[c] - n0v * e0c) * invv
        pltpu.sync_copy(out_v, out_hbm.at[pl.ds(b * D, D)])
        return carry

    lax.fori_loop(0, ROWS_PER_W, row_body, 0)


@functools.partial(
    pl.kernel,
    out_type=jax.ShapeDtypeStruct((B * D,), jnp.float32),
    mesh=plsc.VectorSubcoreMesh(core_axis_name="c", subcore_axis_name="s"),
    compiler_params=pltpu.CompilerParams(
        needs_layout_passes=False, use_tc_tiling_on_sc=False),
    scratch_types=[
        pltpu.VMEM((208,), jnp.int32),
        pltpu.VMEM((L, D), jnp.float32),
        pltpu.VMEM((1, D), jnp.float32),
        pltpu.VMEM((D,), jnp.float32),
        pltpu.SemaphoreType.DMA,
    ],
)
def _sc_encode(tok_hbm, tab_hbm, out_hbm, ids_v, rows_v, e0_v, out_v, sem):
    _tec_kernel(tok_hbm, tab_hbm, out_hbm, ids_v, rows_v, e0_v, out_v, sem)


def kernel(token_ids, embed_weight):
    flat = _sc_encode(token_ids.astype(jnp.int32).reshape(-1), embed_weight)
    return flat.reshape(B, D)


# speedup vs baseline: 26.1202x; 2.1899x over previous
import functools

import jax
import jax.numpy as jnp
from jax import lax
from jax.experimental import pallas as pl
from jax.experimental.pallas import tpu as pltpu
from jax.experimental.pallas import tpu_sc as plsc

D = 64
B = 16384
L = 200
LANES = 16

NC = 2
NS = 16
NW = NC * NS
ROWS_PER_W = B // NW

G0, G1 = 96, 104


def _issue_ids(tok_hbm, b, ids_v, sem):
    pltpu.async_copy(tok_hbm.at[pl.ds(b * L, L)], ids_v.at[pl.ds(0, L)], sem)


def _wait_ids(tok_hbm, b, ids_v, sem):
    pltpu.make_async_copy(
        tok_hbm.at[pl.ds(b * L, L)], ids_v.at[pl.ds(0, L)], sem).wait()


def _issue_gather(tab_hbm, ids_v, rows_v, sem):
    pltpu.async_copy(
        tab_hbm.at[ids_v.at[pl.ds(0, G0)]], rows_v.at[pl.ds(0, G0)], sem)
    pltpu.async_copy(
        tab_hbm.at[ids_v.at[pl.ds(G0, G1)]], rows_v.at[pl.ds(G0, G1)], sem)


def _wait_gather(tab_hbm, ids_v, rows_v, sem):
    pltpu.make_async_copy(
        tab_hbm.at[ids_v.at[pl.ds(0, G0)]], rows_v.at[pl.ds(0, G0)], sem).wait()
    pltpu.make_async_copy(
        tab_hbm.at[ids_v.at[pl.ds(G0, G1)]], rows_v.at[pl.ds(G0, G1)], sem).wait()


def _count_pad(ids_v, lanes):
    zacc = plsc.all_reduce_population_count(ids_v[pl.ds(0, LANES)] == 0)
    for k in range(1, 12):
        zacc = zacc + plsc.all_reduce_population_count(
            ids_v[pl.ds(k * LANES, LANES)] == 0)
    zacc = zacc + plsc.all_reduce_population_count(
        (ids_v[pl.ds(192, LANES)] == 0) & (lanes < 8))
    return zacc.astype(jnp.float32)


def _accumulate(rows_v):
    def acc_body(j, accs):
        a0, a1, a2, a3 = accs
        r0 = j * 8
        for u in range(8):
            r = r0 + u
            a0 = a0 + rows_v[r, pl.ds(0, LANES)]
            a1 = a1 + rows_v[r, pl.ds(16, LANES)]
            a2 = a2 + rows_v[r, pl.ds(32, LANES)]
            a3 = a3 + rows_v[r, pl.ds(48, LANES)]
        return (a0, a1, a2, a3)

    z = jnp.zeros((LANES,), jnp.float32)
    return lax.fori_loop(0, L // 8, acc_body, (z, z, z, z))


def _tec_kernel(tok_hbm, tab_hbm, out_hbm,
                i0, i1, i2, i3, ra, rb, e0_v, oa, ob,
                gs0, gs1, is0, is1, is2, is3, os0, os1):
    ids4 = (i0, i1, i2, i3)
    idsem4 = (is0, is1, is2, is3)
    rows2 = (ra, rb)
    gsem2 = (gs0, gs1)
    outs2 = (oa, ob)
    osem2 = (os0, os1)

    wid = lax.axis_index("s") * NC + lax.axis_index("c")
    base = wid * ROWS_PER_W

    pltpu.sync_copy(tab_hbm.at[pl.ds(0, 1)], e0_v)
    lanes = lax.iota(jnp.int32, LANES)

    _issue_ids(tok_hbm, base, ids4[0], idsem4[0])
    _issue_ids(tok_hbm, base + 1, ids4[1], idsem4[1])
    _wait_ids(tok_hbm, base, ids4[0], idsem4[0])
    _issue_gather(tab_hbm, ids4[0], rows2[0], gsem2[0])

    def quad_body(q, carry):
        for s in range(4):
            i = q * 4 + s
            b = base + i

            @pl.when(i + 2 < ROWS_PER_W)
            def _():
                _issue_ids(tok_hbm, b + 2, ids4[(s + 2) % 4],
                           idsem4[(s + 2) % 4])

            @pl.when(i + 1 < ROWS_PER_W)
            def _():
                _wait_ids(tok_hbm, b + 1, ids4[(s + 1) % 4],
                          idsem4[(s + 1) % 4])
                _issue_gather(tab_hbm, ids4[(s + 1) % 4], rows2[(s + 1) % 2],
                              gsem2[(s + 1) % 2])

            n0v = _count_pad(ids4[s], lanes)
            invv = 1.0 / jnp.maximum(float(L) - n0v, 1.0)

            _wait_gather(tab_hbm, ids4[s], rows2[s % 2], gsem2[s % 2])
            accs = _accumulate(rows2[s % 2])

            @pl.when(i >= 2)
            def _():
                pltpu.make_async_copy(
                    outs2[s % 2],
                    out_hbm.at[pl.ds((b - 2) * D, D)], osem2[s % 2]).wait()

            for c in range(4):
                e0c = e0_v[0, pl.ds(c * LANES, LANES)]
                outs2[s % 2][pl.ds(c * LANES, LANES)] = (
                    (accs[c] - n0v * e0c) * invv)
            pltpu.async_copy(
                outs2[s % 2], out_hbm.at[pl.ds(b * D, D)], osem2[s % 2])
        return carry

    lax.fori_loop(0, ROWS_PER_W // 4, quad_body, 0)

    last = base + ROWS_PER_W - 2
    pltpu.make_async_copy(
        outs2[0], out_hbm.at[pl.ds(last * D, D)], osem2[0]).wait()
    pltpu.make_async_copy(
        outs2[1], out_hbm.at[pl.ds((last + 1) * D, D)], osem2[1]).wait()


@functools.partial(
    pl.kernel,
    out_type=jax.ShapeDtypeStruct((B * D,), jnp.float32),
    mesh=plsc.VectorSubcoreMesh(core_axis_name="c", subcore_axis_name="s"),
    compiler_params=pltpu.CompilerParams(
        needs_layout_passes=False, use_tc_tiling_on_sc=False),
    scratch_types=[
        pltpu.VMEM((208,), jnp.int32),
        pltpu.VMEM((208,), jnp.int32),
        pltpu.VMEM((208,), jnp.int32),
        pltpu.VMEM((208,), jnp.int32),
        pltpu.VMEM((L, D), jnp.float32),
        pltpu.VMEM((L, D), jnp.float32),
        pltpu.VMEM((1, D), jnp.float32),
        pltpu.VMEM((D,), jnp.float32),
        pltpu.VMEM((D,), jnp.float32),
        pltpu.SemaphoreType.DMA,
        pltpu.SemaphoreType.DMA,
        pltpu.SemaphoreType.DMA,
        pltpu.SemaphoreType.DMA,
        pltpu.SemaphoreType.DMA,
        pltpu.SemaphoreType.DMA,
        pltpu.SemaphoreType.DMA,
        pltpu.SemaphoreType.DMA,
    ],
)
def _sc_encode(*args):
    _tec_kernel(*args)


def kernel(token_ids, embed_weight):
    flat = _sc_encode(token_ids.astype(jnp.int32).reshape(-1), embed_weight)
    return flat.reshape(B, D)


# speedup vs baseline: 26.1833x; 1.0024x over previous
import functools

import jax
import jax.numpy as jnp
from jax import lax
from jax.experimental import pallas as pl
from jax.experimental.pallas import tpu as pltpu
from jax.experimental.pallas import tpu_sc as plsc

D = 64
B = 16384
L = 200
LANES = 16

NC = 2
NS = 16
NW = NC * NS
ROWS_PER_W = B // NW

G0, G1 = 96, 104


def _issue_ids(tok_hbm, b, ids_v, sem):
    pltpu.async_copy(tok_hbm.at[pl.ds(b * L, L)], ids_v.at[pl.ds(0, L)], sem)


def _wait_ids(tok_hbm, b, ids_v, sem):
    pltpu.make_async_copy(
        tok_hbm.at[pl.ds(b * L, L)], ids_v.at[pl.ds(0, L)], sem).wait()


def _issue_gather(tab_hbm, ids_v, rows_v, sem):
    pltpu.async_copy(
        tab_hbm.at[ids_v.at[pl.ds(0, G0)]], rows_v.at[pl.ds(0, G0)], sem)
    pltpu.async_copy(
        tab_hbm.at[ids_v.at[pl.ds(G0, G1)]], rows_v.at[pl.ds(G0, G1)], sem)


def _wait_gather(tab_hbm, ids_v, rows_v, sem):
    pltpu.make_async_copy(
        tab_hbm.at[ids_v.at[pl.ds(0, G0)]], rows_v.at[pl.ds(0, G0)], sem).wait()
    pltpu.make_async_copy(
        tab_hbm.at[ids_v.at[pl.ds(G0, G1)]], rows_v.at[pl.ds(G0, G1)], sem).wait()


def _count_pad(ids_v, lanes):
    zacc = plsc.all_reduce_population_count(ids_v[pl.ds(0, LANES)] == 0)
    for k in range(1, 12):
        zacc = zacc + plsc.all_reduce_population_count(
            ids_v[pl.ds(k * LANES, LANES)] == 0)
    zacc = zacc + plsc.all_reduce_population_count(
        (ids_v[pl.ds(192, LANES)] == 0) & (lanes < 8))
    return zacc.astype(jnp.float32)


def _accumulate(rows_v):
    def acc_body(j, accs):
        a0, a1, a2, a3 = accs
        r0 = j * 20
        for u in range(20):
            r = r0 + u
            a0 = a0 + rows_v[r, pl.ds(0, LANES)]
            a1 = a1 + rows_v[r, pl.ds(16, LANES)]
            a2 = a2 + rows_v[r, pl.ds(32, LANES)]
            a3 = a3 + rows_v[r, pl.ds(48, LANES)]
        return (a0, a1, a2, a3)

    z = jnp.zeros((LANES,), jnp.float32)
    return lax.fori_loop(0, L // 20, acc_body, (z, z, z, z))


def _tec_kernel(tok_hbm, tab_hbm, out_hbm,
                i0, i1, i2, i3, ra, rb, e0_v, oa, ob,
                gs0, gs1, is0, is1, is2, is3, os0, os1):
    ids4 = (i0, i1, i2, i3)
    idsem4 = (is0, is1, is2, is3)
    rows2 = (ra, rb)
    gsem2 = (gs0, gs1)
    outs2 = (oa, ob)
    osem2 = (os0, os1)

    wid = lax.axis_index("s") * NC + lax.axis_index("c")
    base = wid * ROWS_PER_W

    pltpu.sync_copy(tab_hbm.at[pl.ds(0, 1)], e0_v)
    lanes = lax.iota(jnp.int32, LANES)

    def row_step(i, s4, s2, do_ids, do_gather, do_outwait):
        b = base + i
        if do_ids:
            _issue_ids(tok_hbm, b + 2, ids4[(s4 + 2) % 4], idsem4[(s4 + 2) % 4])
        if do_gather:
            _wait_ids(tok_hbm, b + 1, ids4[(s4 + 1) % 4], idsem4[(s4 + 1) % 4])
            _issue_gather(tab_hbm, ids4[(s4 + 1) % 4], rows2[(s2 + 1) % 2],
                          gsem2[(s2 + 1) % 2])
        n0v = _count_pad(ids4[s4], lanes)
        invv = 1.0 / jnp.maximum(float(L) - n0v, 1.0)
        _wait_gather(tab_hbm, ids4[s4], rows2[s2], gsem2[s2])
        accs = _accumulate(rows2[s2])
        if do_outwait:
            pltpu.make_async_copy(
                outs2[s2], out_hbm.at[pl.ds((b - 2) * D, D)], osem2[s2]).wait()
        for c in range(4):
            e0c = e0_v[0, pl.ds(c * LANES, LANES)]
            outs2[s2][pl.ds(c * LANES, LANES)] = (accs[c] - n0v * e0c) * invv
        pltpu.async_copy(outs2[s2], out_hbm.at[pl.ds(b * D, D)], osem2[s2])

    _issue_ids(tok_hbm, base, ids4[0], idsem4[0])
    _issue_ids(tok_hbm, base + 1, ids4[1], idsem4[1])
    _wait_ids(tok_hbm, base, ids4[0], idsem4[0])
    _issue_gather(tab_hbm, ids4[0], rows2[0], gsem2[0])
    row_step(0, 0, 0, True, True, False)
    row_step(1, 1, 1, True, True, False)

    def quad_body(q, carry):
        for s in range(4):
            i = q * 4 + 2 + s
            row_step(i, (s + 2) % 4, s % 2, True, True, True)
        return carry

    lax.fori_loop(0, (ROWS_PER_W - 4) // 4, quad_body, 0)

    row_step(ROWS_PER_W - 2, (ROWS_PER_W - 2) % 4, 0, False, True, True)
    row_step(ROWS_PER_W - 1, (ROWS_PER_W - 1) % 4, 1, False, False, True)
    last = base + ROWS_PER_W - 2
    pltpu.make_async_copy(
        outs2[0], out_hbm.at[pl.ds(last * D, D)], osem2[0]).wait()
    pltpu.make_async_copy(
        outs2[1], out_hbm.at[pl.ds((last + 1) * D, D)], osem2[1]).wait()


@functools.partial(
    pl.kernel,
    out_type=jax.ShapeDtypeStruct((B * D,), jnp.float32),
    mesh=plsc.VectorSubcoreMesh(core_axis_name="c", subcore_axis_name="s"),
    compiler_params=pltpu.CompilerParams(
        needs_layout_passes=False, use_tc_tiling_on_sc=False),
    scratch_types=[
        pltpu.VMEM((208,), jnp.int32),
        pltpu.VMEM((208,), jnp.int32),
        pltpu.VMEM((208,), jnp.int32),
        pltpu.VMEM((208,), jnp.int32),
        pltpu.VMEM((L, D), jnp.float32),
        pltpu.VMEM((L, D), jnp.float32),
        pltpu.VMEM((1, D), jnp.float32),
        pltpu.VMEM((D,), jnp.float32),
        pltpu.VMEM((D,), jnp.float32),
        pltpu.SemaphoreType.DMA,
        pltpu.SemaphoreType.DMA,
        pltpu.SemaphoreType.DMA,
        pltpu.SemaphoreType.DMA,
        pltpu.SemaphoreType.DMA,
        pltpu.SemaphoreType.DMA,
        pltpu.SemaphoreType.DMA,
        pltpu.SemaphoreType.DMA,
    ],
)
def _sc_encode(*args):
    _tec_kernel(*args)


def kernel(token_ids, embed_weight):
    flat = _sc_encode(token_ids.astype(jnp.int32).reshape(-1), embed_weight)
    return flat.reshape(B, D)


# speedup vs baseline: 33.8776x; 1.2939x over previous
import functools

import jax
import jax.numpy as jnp
from jax import lax
from jax.experimental import pallas as pl
from jax.experimental.pallas import tpu as pltpu
from jax.experimental.pallas import tpu_sc as plsc

D = 64
B = 16384
L = 200
LANES = 16

NC = 2
NS = 16
NW = NC * NS
ROWS_PER_W = B // NW

G0, G1 = 96, 104

IDS_AHEAD, NIDS = 5, 6
G_AHEAD, NG = 3, 4
UNROLL = 12


def _issue_ids(tok_hbm, b, ids_v, sem):
    pltpu.async_copy(tok_hbm.at[pl.ds(b * L, L)], ids_v.at[pl.ds(0, L)], sem)


def _wait_ids(tok_hbm, b, ids_v, sem):
    pltpu.make_async_copy(
        tok_hbm.at[pl.ds(b * L, L)], ids_v.at[pl.ds(0, L)], sem).wait()


def _issue_gather(tab_hbm, ids_v, rows_v, sem):
    pltpu.async_copy(
        tab_hbm.at[ids_v.at[pl.ds(0, G0)]], rows_v.at[pl.ds(0, G0)], sem)
    pltpu.async_copy(
        tab_hbm.at[ids_v.at[pl.ds(G0, G1)]], rows_v.at[pl.ds(G0, G1)], sem)


def _wait_gather(tab_hbm, ids_v, rows_v, sem):
    pltpu.make_async_copy(
        tab_hbm.at[ids_v.at[pl.ds(0, G0)]], rows_v.at[pl.ds(0, G0)], sem).wait()
    pltpu.make_async_copy(
        tab_hbm.at[ids_v.at[pl.ds(G0, G1)]], rows_v.at[pl.ds(G0, G1)], sem).wait()


def _count_pad(ids_v, lanes):
    zacc = plsc.all_reduce_population_count(ids_v[pl.ds(0, LANES)] == 0)
    for k in range(1, 12):
        zacc = zacc + plsc.all_reduce_population_count(
            ids_v[pl.ds(k * LANES, LANES)] == 0)
    zacc = zacc + plsc.all_reduce_population_count(
        (ids_v[pl.ds(192, LANES)] == 0) & (lanes < 8))
    return zacc.astype(jnp.float32)


def _accumulate(rows_v):
    def acc_body(j, accs):
        a0, a1, a2, a3 = accs
        r0 = j * 20
        for u in range(20):
            r = r0 + u
            a0 = a0 + rows_v[r, pl.ds(0, LANES)]
            a1 = a1 + rows_v[r, pl.ds(16, LANES)]
            a2 = a2 + rows_v[r, pl.ds(32, LANES)]
            a3 = a3 + rows_v[r, pl.ds(48, LANES)]
        return (a0, a1, a2, a3)

    z = jnp.zeros((LANES,), jnp.float32)
    return lax.fori_loop(0, L // 20, acc_body, (z, z, z, z))


def _tec_kernel(tok_hbm, tab_hbm, out_hbm, *scratch):
    ids6 = scratch[0:NIDS]
    rows4 = scratch[NIDS:NIDS + NG]
    e0_v = scratch[NIDS + NG]
    outs2 = scratch[NIDS + NG + 1:NIDS + NG + 3]
    sems = scratch[NIDS + NG + 3:]
    idsem6 = sems[0:NIDS]
    gsem4 = sems[NIDS:NIDS + NG]
    osem2 = sems[NIDS + NG:NIDS + NG + 2]

    wid = lax.axis_index("s") * NC + lax.axis_index("c")
    base = wid * ROWS_PER_W

    pltpu.sync_copy(tab_hbm.at[pl.ds(0, 1)], e0_v)
    lanes = lax.iota(jnp.int32, LANES)

    def row_step(i, m, do_ids, do_gather, do_outwait):
        b = base + i
        if do_ids:
            sl = (m + IDS_AHEAD) % NIDS
            _issue_ids(tok_hbm, b + IDS_AHEAD, ids6[sl], idsem6[sl])
        if do_gather:
            sli = (m + G_AHEAD) % NIDS
            slg = (m + G_AHEAD) % NG
            _wait_ids(tok_hbm, b + G_AHEAD, ids6[sli], idsem6[sli])
            _issue_gather(tab_hbm, ids6[sli], rows4[slg], gsem4[slg])
        n0v = _count_pad(ids6[m % NIDS], lanes)
        invv = 1.0 / jnp.maximum(float(L) - n0v, 1.0)
        _wait_gather(tab_hbm, ids6[m % NIDS], rows4[m % NG], gsem4[m % NG])
        accs = _accumulate(rows4[m % NG])
        if do_outwait:
            pltpu.make_async_copy(
                outs2[m % 2], out_hbm.at[pl.ds((b - 2) * D, D)],
                osem2[m % 2]).wait()
        for c in range(4):
            e0c = e0_v[0, pl.ds(c * LANES, LANES)]
            outs2[m % 2][pl.ds(c * LANES, LANES)] = (accs[c] - n0v * e0c) * invv
        pltpu.async_copy(outs2[m % 2], out_hbm.at[pl.ds(b * D, D)], osem2[m % 2])

    for k in range(IDS_AHEAD):
        _issue_ids(tok_hbm, base + k, ids6[k], idsem6[k])
    for k in range(G_AHEAD):
        _wait_ids(tok_hbm, base + k, ids6[k], idsem6[k])
        _issue_gather(tab_hbm, ids6[k], rows4[k], gsem4[k])
    for k in range(G_AHEAD):
        row_step(k, k, True, True, k >= 2)

    def unroll_body(q, carry):
        for s in range(UNROLL):
            row_step(G_AHEAD + q * UNROLL + s, G_AHEAD + s, True, True, True)
        return carry

    lax.fori_loop(0, (ROWS_PER_W - IDS_AHEAD - G_AHEAD - 1) // UNROLL + 1,
                  unroll_body, 0)

    for i in range(ROWS_PER_W - IDS_AHEAD, ROWS_PER_W):
        row_step(i, i % UNROLL, False, i + G_AHEAD < ROWS_PER_W, True)
    last = base + ROWS_PER_W - 2
    pltpu.make_async_copy(
        outs2[0], out_hbm.at[pl.ds(last * D, D)], osem2[0]).wait()
    pltpu.make_async_copy(
        outs2[1], out_hbm.at[pl.ds((last + 1) * D, D)], osem2[1]).wait()


@functools.partial(
    pl.kernel,
    out_type=jax.ShapeDtypeStruct((B * D,), jnp.float32),
    mesh=plsc.VectorSubcoreMesh(core_axis_name="c", subcore_axis_name="s"),
    compiler_params=pltpu.CompilerParams(
        needs_layout_passes=False, use_tc_tiling_on_sc=False),
    scratch_types=[
        pltpu.VMEM((208,), jnp.int32),
        pltpu.VMEM((208,), jnp.int32),
        pltpu.VMEM((208,), jnp.int32),
        pltpu.VMEM((208,), jnp.int32),
        pltpu.VMEM((208,), jnp.int32),
        pltpu.VMEM((208,), jnp.int32),
        pltpu.VMEM((L, D), jnp.float32),
        pltpu.VMEM((L, D), jnp.float32),
        pltpu.VMEM((L, D), jnp.float32),
        pltpu.VMEM((L, D), jnp.float32),
        pltpu.VMEM((1, D), jnp.float32),
        pltpu.VMEM((D,), jnp.float32),
        pltpu.VMEM((D,), jnp.float32),
        pltpu.SemaphoreType.DMA,
        pltpu.SemaphoreType.DMA,
        pltpu.SemaphoreType.DMA,
        pltpu.SemaphoreType.DMA,
        pltpu.SemaphoreType.DMA,
        pltpu.SemaphoreType.DMA,
        pltpu.SemaphoreType.DMA,
        pltpu.SemaphoreType.DMA,
        pltpu.SemaphoreType.DMA,
        pltpu.SemaphoreType.DMA,
        pltpu.SemaphoreType.DMA,
        pltpu.SemaphoreType.DMA,
    ],
)
def _sc_encode(*args):
    _tec_kernel(*args)


def kernel(token_ids, embed_weight):
    flat = _sc_encode(token_ids.astype(jnp.int32).reshape(-1), embed_weight)
    return flat.reshape(B, D)


# speedup vs baseline: 34.0677x; 1.0056x over previous
import functools

import jax
import jax.numpy as jnp
from jax import lax
from jax.experimental import pallas as pl
from jax.experimental.pallas import tpu as pltpu
from jax.experimental.pallas import tpu_sc as plsc

D = 64
B = 16384
L = 200
LANES = 16

NC = 2
NS = 16
NW = NC * NS
ROWS_PER_W = B // NW

G0, G1 = 96, 104

IDS_AHEAD, NIDS = 5, 6
G_AHEAD, NG = 4, 6
UNROLL = 6
N_STEADY = ((ROWS_PER_W - IDS_AHEAD - G_AHEAD) // UNROLL) * UNROLL
TAIL_START = G_AHEAD + N_STEADY


def _issue_ids(tok_hbm, b, ids_v, sem):
    pltpu.async_copy(tok_hbm.at[pl.ds(b * L, L)], ids_v.at[pl.ds(0, L)], sem)


def _wait_ids(tok_hbm, b, ids_v, sem):
    pltpu.make_async_copy(
        tok_hbm.at[pl.ds(b * L, L)], ids_v.at[pl.ds(0, L)], sem).wait()


def _issue_gather(tab_hbm, ids_v, rows_v, sem):
    pltpu.async_copy(
        tab_hbm.at[ids_v.at[pl.ds(0, G0)]], rows_v.at[pl.ds(0, G0)], sem)
    pltpu.async_copy(
        tab_hbm.at[ids_v.at[pl.ds(G0, G1)]], rows_v.at[pl.ds(G0, G1)], sem)


def _wait_gather(tab_hbm, ids_v, rows_v, sem):
    pltpu.make_async_copy(
        tab_hbm.at[ids_v.at[pl.ds(0, G0)]], rows_v.at[pl.ds(0, G0)], sem).wait()
    pltpu.make_async_copy(
        tab_hbm.at[ids_v.at[pl.ds(G0, G1)]], rows_v.at[pl.ds(G0, G1)], sem).wait()


def _count_pad(ids_v, lanes):
    zacc = plsc.all_reduce_population_count(ids_v[pl.ds(0, LANES)] == 0)
    for k in range(1, 12):
        zacc = zacc + plsc.all_reduce_population_count(
            ids_v[pl.ds(k * LANES, LANES)] == 0)
    zacc = zacc + plsc.all_reduce_population_count(
        (ids_v[pl.ds(192, LANES)] == 0) & (lanes < 8))
    return zacc.astype(jnp.float32)


def _accumulate(rows_v):
    def acc_body(j, accs):
        a0, a1, a2, a3 = accs
        r0 = j * 20
        for u in range(20):
            r = r0 + u
            a0 = a0 + rows_v[r, pl.ds(0, LANES)]
            a1 = a1 + rows_v[r, pl.ds(16, LANES)]
            a2 = a2 + rows_v[r, pl.ds(32, LANES)]
            a3 = a3 + rows_v[r, pl.ds(48, LANES)]
        return (a0, a1, a2, a3)

    z = jnp.zeros((LANES,), jnp.float32)
    return lax.fori_loop(0, L // 20, acc_body, (z, z, z, z))


def _tec_kernel(tok_hbm, tab_hbm, out_hbm, *scratch):
    ids6 = scratch[0:NIDS]
    rows4 = scratch[NIDS:NIDS + NG]
    e0_v = scratch[NIDS + NG]
    outs2 = scratch[NIDS + NG + 1:NIDS + NG + 3]
    sems = scratch[NIDS + NG + 3:]
    idsem6 = sems[0:NIDS]
    gsem4 = sems[NIDS:NIDS + NG]
    osem2 = sems[NIDS + NG:NIDS + NG + 2]

    wid = lax.axis_index("s") * NC + lax.axis_index("c")
    base = wid * ROWS_PER_W

    pltpu.sync_copy(tab_hbm.at[pl.ds(0, 1)], e0_v)
    lanes = lax.iota(jnp.int32, LANES)

    def row_step(i, m, do_ids, do_gather, do_outwait):
        b = base + i
        if do_ids:
            sl = (m + IDS_AHEAD) % NIDS
            _issue_ids(tok_hbm, b + IDS_AHEAD, ids6[sl], idsem6[sl])
        if do_gather:
            sli = (m + G_AHEAD) % NIDS
            slg = (m + G_AHEAD) % NG
            _wait_ids(tok_hbm, b + G_AHEAD, ids6[sli], idsem6[sli])
            _issue_gather(tab_hbm, ids6[sli], rows4[slg], gsem4[slg])
        n0v = _count_pad(ids6[m % NIDS], lanes)
        invv = 1.0 / jnp.maximum(float(L) - n0v, 1.0)
        _wait_gather(tab_hbm, ids6[m % NIDS], rows4[m % NG], gsem4[m % NG])
        accs = _accumulate(rows4[m % NG])
        if do_outwait:
            pltpu.make_async_copy(
                outs2[m % 2], out_hbm.at[pl.ds((b - 2) * D, D)],
                osem2[m % 2]).wait()
        for c in range(4):
            e0c = e0_v[0, pl.ds(c * LANES, LANES)]
            outs2[m % 2][pl.ds(c * LANES, LANES)] = (accs[c] - n0v * e0c) * invv
        pltpu.async_copy(outs2[m % 2], out_hbm.at[pl.ds(b * D, D)], osem2[m % 2])

    for k in range(IDS_AHEAD):
        _issue_ids(tok_hbm, base + k, ids6[k], idsem6[k])
    for k in range(G_AHEAD):
        _wait_ids(tok_hbm, base + k, ids6[k], idsem6[k])
        _issue_gather(tab_hbm, ids6[k], rows4[k], gsem4[k])
    for k in range(G_AHEAD):
        row_step(k, k, True, True, k >= 2)

    def unroll_body(q, carry):
        for s in range(UNROLL):
            row_step(G_AHEAD + q * UNROLL + s, G_AHEAD + s, True, True, True)
        return carry

    lax.fori_loop(0, N_STEADY // UNROLL, unroll_body, 0)

    for i in range(TAIL_START, ROWS_PER_W):
        row_step(i, i % UNROLL, i + IDS_AHEAD < ROWS_PER_W,
                 i + G_AHEAD < ROWS_PER_W, True)
    last = base + ROWS_PER_W - 2
    pltpu.make_async_copy(
        outs2[0], out_hbm.at[pl.ds(last * D, D)], osem2[0]).wait()
    pltpu.make_async_copy(
        outs2[1], out_hbm.at[pl.ds((last + 1) * D, D)], osem2[1]).wait()


@functools.partial(
    pl.kernel,
    out_type=jax.ShapeDtypeStruct((B * D,), jnp.float32),
    mesh=plsc.VectorSubcoreMesh(core_axis_name="c", subcore_axis_name="s"),
    compiler_params=pltpu.CompilerParams(
        needs_layout_passes=False, use_tc_tiling_on_sc=False),
    scratch_types=(
        [pltpu.VMEM((208,), jnp.int32)] * NIDS
        + [pltpu.VMEM((L, D), jnp.float32)] * NG
        + [pltpu.VMEM((1, D), jnp.float32)]
        + [pltpu.VMEM((D,), jnp.float32)] * 2
        + [pltpu.SemaphoreType.DMA] * (NIDS + NG + 2)
    ),
)
def _sc_encode(*args):
    _tec_kernel(*args)


def kernel(token_ids, embed_weight):
    flat = _sc_encode(token_ids.astype(jnp.int32).reshape(-1), embed_weight)
    return flat.reshape(B, D)
